# baseline (speedup 1.0000x reference)
import functools

import jax
import jax.numpy as jnp
from jax import lax
from jax.experimental import pallas as pl
from jax.experimental.pallas import tpu as pltpu
from jax.experimental.pallas import tpu_sc as plsc

_BATCH = 16384
_EMB = 32
_NTAB = 4
_ROWW = 128
_VPR = _ROWW // _EMB
_NC = 2
_NS = 16
_NW = _NC * _NS
_BPW = _BATCH // _NW
_CHUNK = 128
_NCHUNK = _BPW // _CHUNK


def _make_kernel():
  mesh = plsc.VectorSubcoreMesh(core_axis_name="c", subcore_axis_name="s")

  @functools.partial(
      pl.kernel,
      mesh=mesh,
      out_type=jax.ShapeDtypeStruct((_BATCH, _NTAB * _EMB), jnp.float32),
      scratch_types=[
          pltpu.VMEM((_NTAB * _NCHUNK, _CHUNK), jnp.int32),
          pltpu.VMEM((_NTAB * _NCHUNK, _CHUNK), jnp.int32),
          pltpu.VMEM((_NTAB, _CHUNK, _ROWW), jnp.float32),
          pltpu.VMEM((_CHUNK, _NTAB * _EMB), jnp.float32),
          pltpu.SemaphoreType.DMA,
      ],
  )
  def body(idx_hbm, wg, wa, wo, wz, out_hbm, idx_v, q_v, rows_v, out_v,
           gsem):
    wid = lax.axis_index("s") * _NC + lax.axis_index("c")
    base = wid * _BPW
    pltpu.sync_copy(idx_hbm.at[wid], idx_v)
    for r in range(_NTAB * _NCHUNK):
      for v in range(_CHUNK // 16):
        q_v[r, pl.ds(v * 16, 16)] = jax.lax.shift_right_logical(
            idx_v[r, pl.ds(v * 16, 16)], 2)
    tables = (wg, wa, wo, wz)
    for j in range(_NCHUNK):
      copies = [
          pltpu.async_copy(
              tables[t].at[q_v.at[t * _NCHUNK + j]], rows_v.at[t], gsem)
          for t in range(_NTAB)
      ]
      for cp in copies:
        cp.wait()

      def select(g, _, j=j):
        for t in range(_NTAB):
          iv = idx_v[t * _NCHUNK + j, pl.ds(g * 16, 16)]
          for l in range(16):
            off = (iv[l] & (_VPR - 1)) * _EMB
            b = g * 16 + l
            for k in range(_EMB // 16):
              out_v[b, pl.ds(t * _EMB + k * 16, 16)] = (
                  rows_v[t, b, pl.ds(off + k * 16, 16)])
        return ()

      lax.fori_loop(0, _CHUNK // 16, select, ())
      pltpu.sync_copy(out_v, out_hbm.at[pl.ds(base + j * _CHUNK, _CHUNK)])

  return body


_gather_concat = _make_kernel()


def kernel(x, W_gender, W_age, W_occupation, W_zip):
  packed = [w.reshape(-1, _ROWW)
            for w in (W_gender, W_age, W_occupation, W_zip)]
  idx = x.T.reshape(_NTAB, _NW, _NCHUNK, _CHUNK).transpose(1, 0, 2, 3)
  idx = idx.reshape(_NW, _NTAB * _NCHUNK, _CHUNK)
  return _gather_concat(idx, *packed)
